# baseline (device time: 151145 ns/iter reference)
import jax
import jax.numpy as jnp
from jax import lax
from jax.experimental import pallas as pl
from jax.experimental.pallas import tpu as pltpu

N_DEV = 4


def kernel(A, B):
    m, k = A.shape
    k2, n = B.shape

    def body(a_ref, b_ref, out_ref, comm_ref, send_sems, recv_sems):
        my_pos = lax.axis_index("i")
        left = (my_pos - 1) % N_DEV
        right = (my_pos + 1) % N_DEV

        barrier_sem = pltpu.get_barrier_semaphore()
        for nbr in [left, right]:
            pl.semaphore_signal(
                barrier_sem, inc=1,
                device_id=(nbr,), device_id_type=pl.DeviceIdType.MESH,
            )
        pl.semaphore_wait(barrier_sem, 2)

        partial = jnp.dot(a_ref[...], b_ref[...],
                          preferred_element_type=jnp.float32)
        out_ref[...] = partial
        comm_ref[0, :, :] = partial

        for h in range(N_DEV - 1):
            rdma = pltpu.make_async_remote_copy(
                src_ref=comm_ref.at[h],
                dst_ref=comm_ref.at[h + 1],
                send_sem=send_sems.at[h],
                recv_sem=recv_sems.at[h],
                device_id=(right,),
                device_id_type=pl.DeviceIdType.MESH,
            )
            rdma.start()
            rdma.wait()
            out_ref[...] += comm_ref[h + 1, :, :]

        out_ref[...] = jnp.maximum(out_ref[...], 0.0)

    return pl.pallas_call(
        body,
        out_shape=jax.ShapeDtypeStruct((m, n), jnp.float32),
        in_specs=[
            pl.BlockSpec(memory_space=pltpu.VMEM),
            pl.BlockSpec(memory_space=pltpu.VMEM),
        ],
        out_specs=pl.BlockSpec(memory_space=pltpu.VMEM),
        scratch_shapes=[
            pltpu.VMEM((N_DEV, m, n), jnp.float32),
            pltpu.SemaphoreType.DMA((N_DEV - 1,)),
            pltpu.SemaphoreType.DMA((N_DEV - 1,)),
        ],
        compiler_params=pltpu.CompilerParams(collective_id=0),
    )(A, B)


# device time: 53808 ns/iter; 2.8090x vs baseline; 2.8090x over previous
import jax
import jax.numpy as jnp
from jax import lax
from jax.experimental import pallas as pl
from jax.experimental.pallas import tpu as pltpu

N_DEV = 4


def kernel(A, B):
    m, k = A.shape
    _, n = B.shape
    ch = m // N_DEV
    half = n // 2

    def body(a_ref, b_ref, out_ref, p_ref, rs_buf,
             rs_send, rs_recv, ag_send, ag_recv):
        my = lax.axis_index("i")
        left = (my - 1) % N_DEV
        right = (my + 1) % N_DEV

        barrier_sem = pltpu.get_barrier_semaphore()
        for nbr in [left, right]:
            pl.semaphore_signal(
                barrier_sem, inc=1,
                device_id=(nbr,), device_id_type=pl.DeviceIdType.MESH,
            )
        pl.semaphore_wait(barrier_sem, 2)

        p_ref[...] = jnp.dot(a_ref[...], b_ref[...],
                             preferred_element_type=jnp.float32)

        def rows(c):
            return pl.ds((c % N_DEV) * ch, ch)

        cw_cols = pl.ds(0, half)
        ccw_cols = pl.ds(half, half)

        for s in range(N_DEV - 1):
            if s == 0:
                cw_src = p_ref.at[rows(my), cw_cols]
                ccw_src = p_ref.at[rows(my), ccw_cols]
            else:
                cw_src = rs_buf.at[0, s - 1]
                ccw_src = rs_buf.at[1, s - 1]
            cw = pltpu.make_async_remote_copy(
                src_ref=cw_src,
                dst_ref=rs_buf.at[0, s],
                send_sem=rs_send.at[0, s],
                recv_sem=rs_recv.at[0, s],
                device_id=(right,),
                device_id_type=pl.DeviceIdType.MESH,
            )
            ccw = pltpu.make_async_remote_copy(
                src_ref=ccw_src,
                dst_ref=rs_buf.at[1, s],
                send_sem=rs_send.at[1, s],
                recv_sem=rs_recv.at[1, s],
                device_id=(left,),
                device_id_type=pl.DeviceIdType.MESH,
            )
            cw.start()
            ccw.start()
            cw.wait()
            ccw.wait()
            cw_chunk = my - s - 1
            ccw_chunk = my + s + 1
            if s < N_DEV - 2:
                rs_buf[0, s] += p_ref[rows(cw_chunk), cw_cols]
                rs_buf[1, s] += p_ref[rows(ccw_chunk), ccw_cols]
            else:
                out_ref[rows(cw_chunk), cw_cols] = (
                    rs_buf[0, s] + p_ref[rows(cw_chunk), cw_cols])
                out_ref[rows(ccw_chunk), ccw_cols] = (
                    rs_buf[1, s] + p_ref[rows(ccw_chunk), ccw_cols])

        for s in range(N_DEV - 1):
            cw = pltpu.make_async_remote_copy(
                src_ref=out_ref.at[rows(my + 1 - s), cw_cols],
                dst_ref=out_ref.at[rows(my + 1 - s), cw_cols],
                send_sem=ag_send.at[0, s],
                recv_sem=ag_recv.at[0, s],
                device_id=(right,),
                device_id_type=pl.DeviceIdType.MESH,
            )
            ccw = pltpu.make_async_remote_copy(
                src_ref=out_ref.at[rows(my - 1 + s), ccw_cols],
                dst_ref=out_ref.at[rows(my - 1 + s), ccw_cols],
                send_sem=ag_send.at[1, s],
                recv_sem=ag_recv.at[1, s],
                device_id=(left,),
                device_id_type=pl.DeviceIdType.MESH,
            )
            cw.start()
            ccw.start()
            cw.wait()
            ccw.wait()

        out_ref[...] = jnp.maximum(out_ref[...], 0.0)

    return pl.pallas_call(
        body,
        out_shape=jax.ShapeDtypeStruct((m, n), jnp.float32),
        in_specs=[
            pl.BlockSpec(memory_space=pltpu.VMEM),
            pl.BlockSpec(memory_space=pltpu.VMEM),
        ],
        out_specs=pl.BlockSpec(memory_space=pltpu.VMEM),
        scratch_shapes=[
            pltpu.VMEM((m, n), jnp.float32),
            pltpu.VMEM((2, N_DEV - 1, ch, half), jnp.float32),
            pltpu.SemaphoreType.DMA((2, N_DEV - 1)),
            pltpu.SemaphoreType.DMA((2, N_DEV - 1)),
            pltpu.SemaphoreType.DMA((2, N_DEV - 1)),
            pltpu.SemaphoreType.DMA((2, N_DEV - 1)),
        ],
        compiler_params=pltpu.CompilerParams(collective_id=0),
    )(A, B)


# device time: 44555 ns/iter; 3.3923x vs baseline; 1.2077x over previous
import jax
import jax.numpy as jnp
from jax import lax
from jax.experimental import pallas as pl
from jax.experimental.pallas import tpu as pltpu

N_DEV = 4
N_STREAM = 2


def kernel(A, B):
    m, k = A.shape
    _, n = B.shape
    ch = m // N_DEV
    qw = n // (2 * N_STREAM)

    def body(a_ref, b_ref, out_ref, p_ref, rs_buf,
             rs_send, rs_recv, ag_send, ag_recv):
        my = lax.axis_index("i")
        left = (my - 1) % N_DEV
        right = (my + 1) % N_DEV

        barrier_sem = pltpu.get_barrier_semaphore()
        for nbr in [left, right]:
            pl.semaphore_signal(
                barrier_sem, inc=1,
                device_id=(nbr,), device_id_type=pl.DeviceIdType.MESH,
            )
        pl.semaphore_wait(barrier_sem, 2)

        def rows(c):
            return pl.ds((c % N_DEV) * ch, ch)

        def cols(d, q):
            return pl.ds((d * N_STREAM + q) * qw, qw)

        streams = [(0, 0), (1, 0), (0, 1), (1, 1)]

        def peer(d):
            return right if d == 0 else left

        def rs_send_chunk(d, s):
            return my - s if d == 0 else my + s

        def rs_recv_chunk(d, s):
            return my - s - 1 if d == 0 else my + s + 1

        def ag_send_chunk(d, s):
            return my + 1 - s if d == 0 else my - 1 + s

        def rs_rdma(d, q, s):
            src = (p_ref.at[rows(my), cols(d, q)] if s == 0
                   else rs_buf.at[d, q, s - 1])
            return pltpu.make_async_remote_copy(
                src_ref=src,
                dst_ref=rs_buf.at[d, q, s],
                send_sem=rs_send.at[d, q, s],
                recv_sem=rs_recv.at[d, q, s],
                device_id=(peer(d),),
                device_id_type=pl.DeviceIdType.MESH,
            )

        def ag_rdma(d, q, s):
            c = ag_send_chunk(d, s)
            return pltpu.make_async_remote_copy(
                src_ref=out_ref.at[rows(c), cols(d, q)],
                dst_ref=out_ref.at[rows(c), cols(d, q)],
                send_sem=ag_send.at[d, q, s],
                recv_sem=ag_recv.at[d, q, s],
                device_id=(peer(d),),
                device_id_type=pl.DeviceIdType.MESH,
            )

        rdmas = {}

        p_ref[rows(my), :] = jnp.dot(
            a_ref[rows(my), :], b_ref[...],
            preferred_element_type=jnp.float32)
        for d, q in streams:
            r = rdmas[("rs", d, q, 0)] = rs_rdma(d, q, 0)
            r.start()

        for j in (3, 1, 2):
            p_ref[rows(my + j), :] = jnp.dot(
                a_ref[rows(my + j), :], b_ref[...],
                preferred_element_type=jnp.float32)

        for s in range(1, N_DEV - 1):
            for d, q in streams:
                rdmas[("rs", d, q, s - 1)].wait_recv()
                rs_buf[d, q, s - 1] += p_ref[rows(rs_recv_chunk(d, s - 1)),
                                             cols(d, q)]
                r = rdmas[("rs", d, q, s)] = rs_rdma(d, q, s)
                r.start()

        s = N_DEV - 2
        for d, q in streams:
            rdmas[("rs", d, q, s)].wait_recv()
            c = rs_recv_chunk(d, s)
            out_ref[rows(c), cols(d, q)] = jnp.maximum(
                rs_buf[d, q, s] + p_ref[rows(c), cols(d, q)], 0.0)
            r = rdmas[("ag", d, q, 0)] = ag_rdma(d, q, 0)
            r.start()

        for s in range(1, N_DEV - 1):
            for d, q in streams:
                rdmas[("ag", d, q, s - 1)].wait_recv()
                r = rdmas[("ag", d, q, s)] = ag_rdma(d, q, s)
                r.start()
        for d, q in streams:
            rdmas[("ag", d, q, N_DEV - 2)].wait_recv()

        for key, r in rdmas.items():
            r.wait_send()

    return pl.pallas_call(
        body,
        out_shape=jax.ShapeDtypeStruct((m, n), jnp.float32),
        in_specs=[
            pl.BlockSpec(memory_space=pltpu.VMEM),
            pl.BlockSpec(memory_space=pltpu.VMEM),
        ],
        out_specs=pl.BlockSpec(memory_space=pltpu.VMEM),
        scratch_shapes=[
            pltpu.VMEM((m, n), jnp.float32),
            pltpu.VMEM((2, N_STREAM, N_DEV - 1, ch, qw), jnp.float32),
            pltpu.SemaphoreType.DMA((2, N_STREAM, N_DEV - 1)),
            pltpu.SemaphoreType.DMA((2, N_STREAM, N_DEV - 1)),
            pltpu.SemaphoreType.DMA((2, N_STREAM, N_DEV - 1)),
            pltpu.SemaphoreType.DMA((2, N_STREAM, N_DEV - 1)),
        ],
        compiler_params=pltpu.CompilerParams(collective_id=0),
    )(A, B)
